# Initial kernel scaffold; baseline (speedup 1.0000x reference)
import jax
import jax.numpy as jnp
from jax.experimental import pallas as pl


def kernel(features, edge_index, W0, b0, W1, b1):
    raise NotImplementedError("write your pallas kernel here")


# speedup vs baseline: 5.5726x; 5.5726x over previous
import functools

import jax
import jax.numpy as jnp
from jax import lax
from jax.experimental import pallas as pl
from jax.experimental.pallas import tpu as pltpu
from jax.experimental.pallas import tpu_sc as plsc

NC = 2
NS = 16
NW = NC * NS

_mesh = functools.partial(
    plsc.VectorSubcoreMesh, core_axis_name="c", subcore_axis_name="s",
    num_cores=NC, num_subcores=NS,
)


def _deg_kernel(NP, E):
    EP = E // NW

    @functools.partial(
        pl.kernel,
        out_type=(
            jax.ShapeDtypeStruct((NW, NP), jnp.float32),
            jax.ShapeDtypeStruct((NW, NP), jnp.float32),
        ),
        mesh=_mesh(),
        compiler_params=pltpu.CompilerParams(needs_layout_passes=False),
        scratch_types=[
            pltpu.VMEM((EP,), jnp.int32),
            pltpu.VMEM((EP,), jnp.int32),
            pltpu.VMEM((NP,), jnp.float32),
            pltpu.VMEM((NP,), jnp.float32),
        ],
    )
    def k(src_hbm, dst_hbm, dout_hbm, din_hbm, src_v, dst_v, dout_v, din_v):
        c = lax.axis_index("c")
        s = lax.axis_index("s")
        w = c * NS + s

        def zero(i, carry):
            dout_v[pl.ds(i * 16, 16)] = jnp.zeros((16,), jnp.float32)
            din_v[pl.ds(i * 16, 16)] = jnp.zeros((16,), jnp.float32)
            return carry

        lax.fori_loop(0, NP // 16, zero, 0)
        pltpu.sync_copy(src_hbm.at[pl.ds(w * EP, EP)], src_v)
        pltpu.sync_copy(dst_hbm.at[pl.ds(w * EP, EP)], dst_v)
        ones = jnp.ones((16,), jnp.float32)

        def body(i, carry):
            plsc.addupdate_scatter(dout_v, [src_v[pl.ds(i * 16, 16)]], ones)
            plsc.addupdate_scatter(din_v, [dst_v[pl.ds(i * 16, 16)]], ones)
            return carry

        lax.fori_loop(0, EP // 16, body, 0)
        pltpu.sync_copy(dout_v, dout_hbm.at[w])
        pltpu.sync_copy(din_v, din_hbm.at[w])

    return k


def _agg_kernel(NP, E, D, K):
    EP = E // NW
    NITER = EP // K
    RP = NP // NS

    @functools.partial(
        pl.kernel,
        out_type=jax.ShapeDtypeStruct((NC, NP, D), jnp.float32),
        mesh=_mesh(),
        scratch_types=[
            pltpu.VMEM((K,), jnp.int32),
            pltpu.VMEM((K,), jnp.int32),
            pltpu.VMEM((K, D), jnp.float32),
            pltpu.VMEM_SHARED((NP, D), jnp.float32),
            pltpu.SemaphoreType.DMA,
        ],
    )
    def k(h_hbm, src_hbm, dst_hbm, zeros_hbm, out_hbm,
          sidx, didx, rows, agg_sh, sem):
        c = lax.axis_index("c")
        s = lax.axis_index("s")
        w = c * NS + s
        pltpu.sync_copy(zeros_hbm.at[pl.ds(s * RP, RP)], agg_sh.at[pl.ds(s * RP, RP)])
        plsc.subcore_barrier()

        base = w * EP

        def body(i, carry):
            off = base + i * K
            pltpu.sync_copy(src_hbm.at[pl.ds(off, K)], sidx)
            pltpu.sync_copy(dst_hbm.at[pl.ds(off, K)], didx)
            pltpu.async_copy(h_hbm.at[sidx], rows, sem).wait()
            pltpu.sync_copy(rows, agg_sh.at[didx], add=True)
            return carry

        lax.fori_loop(0, NITER, body, 0)
        plsc.subcore_barrier()
        pltpu.sync_copy(agg_sh.at[pl.ds(s * RP, RP)], out_hbm.at[c, pl.ds(s * RP, RP)])

    return k


def _norm_from(degp_ref):
    d = lax.dot_general(degp_ref[...], jnp.ones((NW, 1), jnp.float32),
                        (((0,), (0,)), ((), ())),
                        preferred_element_type=jnp.float32)
    return lax.rsqrt(jnp.maximum(d, 1.0))


def _prescale_body(x_ref, degp_ref, o_ref):
    o_ref[...] = x_ref[...] * _norm_from(degp_ref)


def _layer_body(aggp_ref, din_ref, dout_ref, w_ref, b_ref, o_ref, *, relu_scale):
    a = aggp_ref[0] + aggp_ref[1]
    a = a * _norm_from(din_ref)
    y = jnp.dot(a, w_ref[...], preferred_element_type=jnp.float32) + b_ref[...]
    if relu_scale:
        y = jnp.maximum(y, 0.0) * _norm_from(dout_ref)
    o_ref[...] = y


def kernel(features, edge_index, W0, b0, W1, b1):
    N, D = features.shape
    H = W0.shape[1]
    E = edge_index.shape[1]
    K = 80
    NP = -(-N // (NS * 8)) * (NS * 8)

    src = edge_index[0]
    dst = edge_index[1]
    zerosD = jnp.zeros((NP, D), jnp.float32)

    deg_out_p, deg_in_p = _deg_kernel(NP, E)(src, dst)

    R = 512
    G = -(-N // R)

    prescale = pl.pallas_call(
        _prescale_body,
        grid=(G,),
        in_specs=[
            pl.BlockSpec((R, D), lambda i: (i, 0)),
            pl.BlockSpec((NW, R), lambda i: (0, i)),
        ],
        out_specs=pl.BlockSpec((R, D), lambda i: (i, 0)),
        out_shape=jax.ShapeDtypeStruct((N, D), jnp.float32),
    )
    hn0 = prescale(features, deg_out_p)

    agg = _agg_kernel(NP, E, D, K)
    agg0 = agg(hn0, src, dst, zerosD)

    def layer(aggp, W, b, relu_scale):
        return pl.pallas_call(
            functools.partial(_layer_body, relu_scale=relu_scale),
            grid=(G,),
            in_specs=[
                pl.BlockSpec((NC, R, D), lambda i: (0, i, 0)),
                pl.BlockSpec((NW, R), lambda i: (0, i)),
                pl.BlockSpec((NW, R), lambda i: (0, i)),
                pl.BlockSpec((D, H), lambda i: (0, 0)),
                pl.BlockSpec((1, H), lambda i: (0, 0)),
            ],
            out_specs=pl.BlockSpec((R, H), lambda i: (i, 0)),
            out_shape=jax.ShapeDtypeStruct((N, H), jnp.float32),
        )(aggp, deg_in_p, deg_out_p, W, b.reshape(1, H))

    h1n = layer(agg0, W0, b0, True)
    agg1 = agg(h1n, src, dst, zerosD)
    out = layer(agg1, W1, b1, False)
    return out
